# baseline (device time: 127845 ns/iter reference)
import jax
import jax.numpy as jnp
from jax import lax
from jax.experimental import pallas as pl
from jax.experimental.pallas import tpu as pltpu

B, S, H, Dh, Dr = 4, 256, 32, 128, 64
D = 4096
DCH = 128
SCALE = (Dh + Dr) ** -0.5
BF = jnp.bfloat16
F32 = jnp.float32
MESH = pl.DeviceIdType.MESH


def _mm(a, w, out_dtype=BF, bn=512, bk=1024):
    m, kd = a.shape
    n = w.shape[1]
    bn = min(bn, n)
    bk = min(bk, kd)
    nk = kd // bk

    def body(a_ref, w_ref, o_ref, acc_ref):
        ki = pl.program_id(1)

        @pl.when(ki == 0)
        def _():
            acc_ref[...] = jnp.zeros_like(acc_ref)

        acc_ref[...] += jnp.dot(
            a_ref[...].astype(BF), w_ref[...].astype(BF),
            preferred_element_type=F32)

        @pl.when(ki == nk - 1)
        def _():
            o_ref[...] = acc_ref[...].astype(o_ref.dtype)

    return pl.pallas_call(
        body,
        grid=(n // bn, nk),
        in_specs=[
            pl.BlockSpec((m, bk), lambda ni, ki: (0, ki)),
            pl.BlockSpec((bk, bn), lambda ni, ki: (ki, ni)),
        ],
        out_specs=pl.BlockSpec((m, bn), lambda ni, ki: (0, ni)),
        out_shape=jax.ShapeDtypeStruct((m, n), out_dtype),
        scratch_shapes=[pltpu.VMEM((m, bn), F32)],
    )(a, w)


def _q_mm_exchange(x, wdkv, wuk, wuv, wkr, wq, bn=1024, bk=2048):
    NB = D // bn
    nk = D // bk

    def body(x_hbm, wdkv_ref, wuk_ref, wuv_ref, wkr_ref, w_ref,
             q_ref, cm_ref, cp_ref, wukp_ref, wuvp_ref, xmbf_ref,
             kr_ref, acc_ref, xm_ref, xp_ref, cma_ref, csa_ref,
             kra_ref, cs_ref, wuks_ref, wuvs_ref,
             copy_sems, send_sems, recv_sems):
        ni = pl.program_id(0)
        ki = pl.program_id(1)
        my_x = lax.axis_index("x")
        my_y = lax.axis_index("y")
        ypeer = (my_x, 1 - my_y)
        mb = 2 * my_x + my_y
        pb = 2 * my_x + 1 - my_y

        @pl.when(jnp.logical_and(ni == 0, ki == 0))
        def _():
            cp_m = pltpu.make_async_copy(x_hbm.at[mb], xm_ref,
                                         copy_sems.at[0])
            cp_p = pltpu.make_async_copy(x_hbm.at[pb], xp_ref,
                                         copy_sems.at[1])
            cp_m.start()
            cp_p.start()
            bar = pltpu.get_barrier_semaphore()
            pl.semaphore_signal(bar, inc=1, device_id=ypeer,
                                device_id_type=MESH)
            pl.semaphore_wait(bar, 1)
            wuks_ref[...] = wuk_ref[...].astype(BF)
            wuvs_ref[...] = wuv_ref[...].astype(BF)
            pltpu.make_async_remote_copy(
                wuks_ref, wukp_ref, send_sems.at[1], recv_sems.at[1],
                device_id=ypeer, device_id_type=MESH).start()
            pltpu.make_async_remote_copy(
                wuvs_ref, wuvp_ref, send_sems.at[2], recv_sems.at[2],
                device_id=ypeer, device_id_type=MESH).start()
            cma_ref[...] = jnp.zeros_like(cma_ref)
            csa_ref[...] = jnp.zeros_like(csa_ref)
            kra_ref[...] = jnp.zeros_like(kra_ref)
            cp_m.wait()
            cp_p.wait()

        @pl.when(ki == 0)
        def _():
            acc_ref[...] = jnp.zeros_like(acc_ref)

        kcols = pl.ds(ki * bk, bk)
        a_bf = xm_ref[:, kcols].astype(BF)
        xmbf_ref[...] = a_bf
        acc_ref[...] += jnp.dot(a_bf, w_ref[...].astype(BF),
                                preferred_element_type=F32)

        @pl.when(ni == 0)
        def _():
            wdkv_bf = wdkv_ref[...].astype(BF)
            cma_ref[...] += jnp.dot(a_bf, wdkv_bf,
                                    preferred_element_type=F32)
            csa_ref[...] += jnp.dot(xp_ref[:, kcols].astype(BF), wdkv_bf,
                                    preferred_element_type=F32)
            kra_ref[...] += jnp.dot(a_bf, wkr_ref[...].astype(BF),
                                    preferred_element_type=F32)

        @pl.when(jnp.logical_and(ni == 0, ki == nk - 1))
        def _():
            cm_ref[...] = cma_ref[...].astype(BF)
            cs_ref[...] = csa_ref[...].astype(BF)
            kr_ref[...] = kra_ref[...].astype(BF)
            pltpu.make_async_remote_copy(
                cs_ref, cp_ref, send_sems.at[0], recv_sems.at[0],
                device_id=ypeer, device_id_type=MESH).start()

        @pl.when(ki == nk - 1)
        def _():
            q_ref[...] = acc_ref[...].astype(BF)

        @pl.when(jnp.logical_and(ni == NB - 1, ki == nk - 1))
        def _():
            pairs = ((cs_ref, cp_ref), (wuks_ref, wukp_ref),
                     (wuvs_ref, wuvp_ref))
            for idx, (src, dst) in enumerate(pairs):
                pltpu.make_async_remote_copy(
                    src, dst, send_sems.at[idx], recv_sems.at[idx],
                    device_id=ypeer, device_id_type=MESH).wait()

    return pl.pallas_call(
        body,
        grid=(NB, nk),
        in_specs=[
            pl.BlockSpec(memory_space=pl.ANY),
            pl.BlockSpec((bk, DCH), lambda ni, ki: (ki, 0)),
            pl.BlockSpec((DCH, D), lambda ni, ki: (0, 0)),
            pl.BlockSpec((DCH, D), lambda ni, ki: (0, 0)),
            pl.BlockSpec((bk, Dr), lambda ni, ki: (ki, 0)),
            pl.BlockSpec((bk, bn), lambda ni, ki: (ki, ni)),
        ],
        out_specs=(
            pl.BlockSpec((S, bn), lambda ni, ki: (0, ni)),
            pl.BlockSpec((S, DCH), lambda ni, ki: (0, 0)),
            pl.BlockSpec((S, DCH), lambda ni, ki: (0, 0)),
            pl.BlockSpec((DCH, D), lambda ni, ki: (0, 0)),
            pl.BlockSpec((DCH, D), lambda ni, ki: (0, 0)),
            pl.BlockSpec((S, bk), lambda ni, ki: (0, ki)),
            pl.BlockSpec((S, Dr), lambda ni, ki: (0, 0)),
        ),
        out_shape=(
            jax.ShapeDtypeStruct((S, D), BF),
            jax.ShapeDtypeStruct((S, DCH), BF),
            jax.ShapeDtypeStruct((S, DCH), BF),
            jax.ShapeDtypeStruct((DCH, D), BF),
            jax.ShapeDtypeStruct((DCH, D), BF),
            jax.ShapeDtypeStruct((S, D), BF),
            jax.ShapeDtypeStruct((S, Dr), BF),
        ),
        scratch_shapes=[
            pltpu.VMEM((S, bn), F32),
            pltpu.VMEM((S, D), F32),
            pltpu.VMEM((S, D), F32),
            pltpu.VMEM((S, DCH), F32),
            pltpu.VMEM((S, DCH), F32),
            pltpu.VMEM((S, Dr), F32),
            pltpu.VMEM((S, DCH), BF),
            pltpu.VMEM((DCH, D), BF),
            pltpu.VMEM((DCH, D), BF),
            pltpu.SemaphoreType.DMA((2,)),
            pltpu.SemaphoreType.DMA((3,)),
            pltpu.SemaphoreType.DMA((3,)),
        ],
        compiler_params=pltpu.CompilerParams(collective_id=0),
    )(x, wdkv, wuk, wuv, wkr, wq)


def _attention(xm_bf, q, kr, cm, cp, wqr, wuk, wuv, wukp, wuvp):
    HG = 16
    CC = (((1,), (1,)), ((), ()))

    def body(x_ref, q_ref, kr_ref, cm_ref, cp_ref, wqr_ref,
             wuk_ref, wuv_ref, wukp_ref, wuvp_ref, o_ref,
             kg_ref, vg_ref):
        x_bf = x_ref[...]
        krm = kr_ref[...]
        qr_g = jnp.dot(x_bf, wqr_ref[...].astype(BF),
                       preferred_element_type=F32).astype(BF)
        cmv = cm_ref[...]
        cpv = cp_ref[...]
        kg_ref[...] = (
            jnp.dot(cmv, wuk_ref[...].astype(BF),
                    preferred_element_type=F32)
            + jnp.dot(cpv, wukp_ref[...], preferred_element_type=F32)
        ).astype(BF)
        vg_ref[...] = (
            jnp.dot(cmv, wuv_ref[...].astype(BF),
                    preferred_element_type=F32)
            + jnp.dot(cpv, wuvp_ref[...], preferred_element_type=F32)
        ).astype(BF)
        for i in range(HG):
            sl = slice(i * Dh, (i + 1) * Dh)
            s = lax.dot_general(q_ref[:, sl], kg_ref[:, sl], CC,
                                preferred_element_type=F32)
            s += lax.dot_general(qr_g[:, i * Dr:(i + 1) * Dr], krm, CC,
                                 preferred_element_type=F32)
            s *= SCALE
            mx = jnp.max(s, axis=-1, keepdims=True)
            p = jnp.exp(s - mx)
            den = jnp.sum(p, axis=-1, keepdims=True)
            o = jnp.dot(p.astype(BF), vg_ref[:, sl],
                        preferred_element_type=F32)
            o_ref[:, sl] = (o / den).astype(BF)

    return pl.pallas_call(
        body,
        grid=(H // HG,),
        in_specs=[
            pl.BlockSpec((S, D), lambda g: (0, 0)),
            pl.BlockSpec((S, HG * Dh), lambda g: (0, g)),
            pl.BlockSpec((S, Dr), lambda g: (0, 0)),
            pl.BlockSpec((S, DCH), lambda g: (0, 0)),
            pl.BlockSpec((S, DCH), lambda g: (0, 0)),
            pl.BlockSpec((D, HG * Dr), lambda g: (0, g)),
            pl.BlockSpec((DCH, HG * Dh), lambda g: (0, g)),
            pl.BlockSpec((DCH, HG * Dh), lambda g: (0, g)),
            pl.BlockSpec((DCH, HG * Dh), lambda g: (0, g)),
            pl.BlockSpec((DCH, HG * Dh), lambda g: (0, g)),
        ],
        out_specs=pl.BlockSpec((S, HG * Dh), lambda g: (0, g)),
        out_shape=jax.ShapeDtypeStruct((S, H * Dh), BF),
        scratch_shapes=[
            pltpu.VMEM((S, HG * Dh), BF),
            pltpu.VMEM((S, HG * Dh), BF),
        ],
        compiler_params=pltpu.CompilerParams(
            vmem_limit_bytes=100 * 1024 * 1024),
    )(xm_bf, q, kr, cm, cp, wqr, wuk, wuv, wukp, wuvp)


def _wo_mm_ag(o_b, wo, bn=1024, bk=1024):
    NB = D // bn
    nk = D // bk

    def body(a_ref, w_ref, out_ref, acc_ref, sbuf, rby, rbx, rbd,
             sa, ra, sb, rb, sc, rc):
        ni = pl.program_id(0)
        ki = pl.program_id(1)
        my_x = lax.axis_index("x")
        my_y = lax.axis_index("y")
        ypeer = (my_x, 1 - my_y)
        xpeer = (1 - my_x, my_y)
        @pl.when(jnp.logical_and(ni == 0, ki == 0))
        def _():
            bar = pltpu.get_barrier_semaphore()
            for p in (ypeer, xpeer):
                pl.semaphore_signal(bar, inc=1, device_id=p,
                                    device_id_type=MESH)
            pl.semaphore_wait(bar, 2)

        @pl.when(ki == 0)
        def _():
            acc_ref[...] = jnp.zeros_like(acc_ref)

        acc_ref[...] += jnp.dot(a_ref[...], w_ref[...].astype(BF),
                                preferred_element_type=F32)

        def desc_a(j):
            jc = pl.ds(j * bn, bn)
            return pltpu.make_async_remote_copy(
                sbuf.at[:, jc], rby.at[:, jc], sa.at[j], ra.at[j],
                device_id=ypeer, device_id_type=MESH)

        def desc_b(j):
            jc = pl.ds(j * bn, bn)
            return pltpu.make_async_remote_copy(
                sbuf.at[:, jc], rbx.at[:, jc], sb.at[j], rb.at[j],
                device_id=xpeer, device_id_type=MESH)

        def desc_c(j):
            jc = pl.ds(j * bn, bn)
            if j % 2 == 0:
                return pltpu.make_async_remote_copy(
                    rby.at[:, jc], rbd.at[:, jc], sc.at[j], rc.at[j],
                    device_id=xpeer, device_id_type=MESH)
            return pltpu.make_async_remote_copy(
                rbx.at[:, jc], rbd.at[:, jc], sc.at[j], rc.at[j],
                device_id=ypeer, device_id_type=MESH)

        def fwd(j):
            (desc_a(j) if j % 2 == 0 else desc_b(j)).wait_recv()
            desc_c(j).start()

        @pl.when(ki == nk - 1)
        def _():
            cols = pl.ds(ni * bn, bn)
            blk = acc_ref[...]
            sbuf[:, cols] = blk.astype(BF)
            out_ref[pl.ds(2 * my_x + my_y, 1), :, cols] = blk[None]

        for jj in range(NB):
            @pl.when(jnp.logical_and(ni == jj, ki == nk - 1))
            def _(jj=jj):
                desc_a(jj).start()
                desc_b(jj).start()
                if jj > 0:
                    fwd(jj - 1)

        @pl.when(jnp.logical_and(ni == NB - 1, ki == nk - 1))
        def _():
            fwd(NB - 1)
            for j in range(NB):
                for d in (desc_a(j), desc_b(j), desc_c(j)):
                    d.wait_send()
                (desc_b(j) if j % 2 == 0 else desc_a(j)).wait_recv()
                desc_c(j).wait_recv()
            out_ref[pl.ds(2 * my_x + 1 - my_y, 1)] = (
                rby[...].astype(F32)[None])
            out_ref[pl.ds(2 * (1 - my_x) + my_y, 1)] = (
                rbx[...].astype(F32)[None])
            out_ref[pl.ds(2 * (1 - my_x) + 1 - my_y, 1)] = (
                rbd[...].astype(F32)[None])

    return pl.pallas_call(
        body,
        grid=(NB, nk),
        in_specs=[
            pl.BlockSpec((S, bk), lambda ni, ki: (0, ki)),
            pl.BlockSpec((bk, bn), lambda ni, ki: (ki, ni)),
        ],
        out_specs=pl.BlockSpec((B, S, D), lambda ni, ki: (0, 0, 0)),
        out_shape=jax.ShapeDtypeStruct((B, S, D), F32),
        scratch_shapes=[
            pltpu.VMEM((S, bn), F32),
            pltpu.VMEM((S, D), BF),
            pltpu.VMEM((S, D), BF),
            pltpu.VMEM((S, D), BF),
            pltpu.VMEM((S, D), BF),
            pltpu.SemaphoreType.DMA((NB,)),
            pltpu.SemaphoreType.DMA((NB,)),
            pltpu.SemaphoreType.DMA((NB,)),
            pltpu.SemaphoreType.DMA((NB,)),
            pltpu.SemaphoreType.DMA((NB,)),
            pltpu.SemaphoreType.DMA((NB,)),
        ],
        compiler_params=pltpu.CompilerParams(collective_id=1),
    )(o_b, wo)


def kernel(x, Wdkv, Wuk, Wuv, Wq, Wqr, Wkr, Wo):
    q, cm, cp, wukp, wuvp, xmbf, kr = _q_mm_exchange(
        x, Wdkv, Wuk, Wuv, Wkr, Wq)
    o = _attention(xmbf, q, kr, cm, cp, Wqr, Wuk, Wuv, wukp, wuvp)
    return _wo_mm_ag(o, Wo, bn=512, bk=2048)


# device time: 110294 ns/iter; 1.1591x vs baseline; 1.1591x over previous
import jax
import jax.numpy as jnp
from jax import lax
from jax.experimental import pallas as pl
from jax.experimental.pallas import tpu as pltpu

B, S, H, Dh, Dr = 4, 256, 32, 128, 64
D = 4096
DCH = 128
SCALE = (Dh + Dr) ** -0.5
BF = jnp.bfloat16
F32 = jnp.float32
MESH = pl.DeviceIdType.MESH


def _mm(a, w, out_dtype=BF, bn=512, bk=1024):
    m, kd = a.shape
    n = w.shape[1]
    bn = min(bn, n)
    bk = min(bk, kd)
    nk = kd // bk

    def body(a_ref, w_ref, o_ref, acc_ref):
        ki = pl.program_id(1)

        @pl.when(ki == 0)
        def _():
            acc_ref[...] = jnp.zeros_like(acc_ref)

        acc_ref[...] += jnp.dot(
            a_ref[...].astype(BF), w_ref[...].astype(BF),
            preferred_element_type=F32)

        @pl.when(ki == nk - 1)
        def _():
            o_ref[...] = acc_ref[...].astype(o_ref.dtype)

    return pl.pallas_call(
        body,
        grid=(n // bn, nk),
        in_specs=[
            pl.BlockSpec((m, bk), lambda ni, ki: (0, ki)),
            pl.BlockSpec((bk, bn), lambda ni, ki: (ki, ni)),
        ],
        out_specs=pl.BlockSpec((m, bn), lambda ni, ki: (0, ni)),
        out_shape=jax.ShapeDtypeStruct((m, n), out_dtype),
        scratch_shapes=[pltpu.VMEM((m, bn), F32)],
    )(a, w)


def _q_mm_exchange(x, wdkv, wuk, wuv, wkr, wq, bn=1024, bk=2048):
    NB = D // bn
    nk = D // bk

    def body(x_hbm, wdkv_ref, wuk_ref, wuv_ref, wkr_ref, w_ref,
             q_ref, cm_ref, cp_ref, wukp_ref, wuvp_ref, xmbf_ref,
             kr_ref, acc_ref, xm_ref, xp_ref, cma_ref, csa_ref,
             kra_ref, cs_ref, wuks_ref, wuvs_ref,
             copy_sems, send_sems, recv_sems):
        ni = pl.program_id(0)
        ki = pl.program_id(1)
        my_x = lax.axis_index("x")
        my_y = lax.axis_index("y")
        ypeer = (my_x, 1 - my_y)
        mb = 2 * my_x + my_y
        pb = 2 * my_x + 1 - my_y

        @pl.when(jnp.logical_and(ni == 0, ki == 0))
        def _():
            cp_m = pltpu.make_async_copy(x_hbm.at[mb], xm_ref,
                                         copy_sems.at[0])
            cp_p = pltpu.make_async_copy(x_hbm.at[pb], xp_ref,
                                         copy_sems.at[1])
            cp_m.start()
            cp_p.start()
            bar = pltpu.get_barrier_semaphore()
            pl.semaphore_signal(bar, inc=1, device_id=ypeer,
                                device_id_type=MESH)
            pl.semaphore_wait(bar, 1)
            wuks_ref[...] = wuk_ref[...].astype(BF)
            wuvs_ref[...] = wuv_ref[...].astype(BF)
            pltpu.make_async_remote_copy(
                wuks_ref, wukp_ref, send_sems.at[1], recv_sems.at[1],
                device_id=ypeer, device_id_type=MESH).start()
            pltpu.make_async_remote_copy(
                wuvs_ref, wuvp_ref, send_sems.at[2], recv_sems.at[2],
                device_id=ypeer, device_id_type=MESH).start()
            cma_ref[...] = jnp.zeros_like(cma_ref)
            csa_ref[...] = jnp.zeros_like(csa_ref)
            kra_ref[...] = jnp.zeros_like(kra_ref)
            cp_m.wait()
            cp_p.wait()

        @pl.when(ki == 0)
        def _():
            acc_ref[...] = jnp.zeros_like(acc_ref)

        kcols = pl.ds(ki * bk, bk)
        a_bf = xm_ref[:, kcols].astype(BF)
        xmbf_ref[...] = a_bf
        acc_ref[...] += jnp.dot(a_bf, w_ref[...].astype(BF),
                                preferred_element_type=F32)

        @pl.when(ni == 0)
        def _():
            wdkv_bf = wdkv_ref[...].astype(BF)
            cma_ref[...] += jnp.dot(a_bf, wdkv_bf,
                                    preferred_element_type=F32)
            csa_ref[...] += jnp.dot(xp_ref[:, kcols].astype(BF), wdkv_bf,
                                    preferred_element_type=F32)
            kra_ref[...] += jnp.dot(a_bf, wkr_ref[...].astype(BF),
                                    preferred_element_type=F32)

        @pl.when(jnp.logical_and(ni == 0, ki == nk - 1))
        def _():
            cm_ref[...] = cma_ref[...].astype(BF)
            cs_ref[...] = csa_ref[...].astype(BF)
            kr_ref[...] = kra_ref[...].astype(BF)
            pltpu.make_async_remote_copy(
                cs_ref, cp_ref, send_sems.at[0], recv_sems.at[0],
                device_id=ypeer, device_id_type=MESH).start()

        @pl.when(ki == nk - 1)
        def _():
            q_ref[...] = acc_ref[...].astype(BF)

        @pl.when(jnp.logical_and(ni == NB - 1, ki == nk - 1))
        def _():
            pairs = ((cs_ref, cp_ref), (wuks_ref, wukp_ref),
                     (wuvs_ref, wuvp_ref))
            for idx, (src, dst) in enumerate(pairs):
                pltpu.make_async_remote_copy(
                    src, dst, send_sems.at[idx], recv_sems.at[idx],
                    device_id=ypeer, device_id_type=MESH).wait()

    return pl.pallas_call(
        body,
        grid=(NB, nk),
        in_specs=[
            pl.BlockSpec(memory_space=pl.ANY),
            pl.BlockSpec((bk, DCH), lambda ni, ki: (ki, 0)),
            pl.BlockSpec((DCH, D), lambda ni, ki: (0, 0)),
            pl.BlockSpec((DCH, D), lambda ni, ki: (0, 0)),
            pl.BlockSpec((bk, Dr), lambda ni, ki: (ki, 0)),
            pl.BlockSpec((bk, bn), lambda ni, ki: (ki, ni)),
        ],
        out_specs=(
            pl.BlockSpec((S, bn), lambda ni, ki: (0, ni)),
            pl.BlockSpec((S, DCH), lambda ni, ki: (0, 0)),
            pl.BlockSpec((S, DCH), lambda ni, ki: (0, 0)),
            pl.BlockSpec((DCH, D), lambda ni, ki: (0, 0)),
            pl.BlockSpec((DCH, D), lambda ni, ki: (0, 0)),
            pl.BlockSpec((S, bk), lambda ni, ki: (0, ki)),
            pl.BlockSpec((S, Dr), lambda ni, ki: (0, 0)),
        ),
        out_shape=(
            jax.ShapeDtypeStruct((S, D), BF),
            jax.ShapeDtypeStruct((S, DCH), BF),
            jax.ShapeDtypeStruct((S, DCH), BF),
            jax.ShapeDtypeStruct((DCH, D), BF),
            jax.ShapeDtypeStruct((DCH, D), BF),
            jax.ShapeDtypeStruct((S, D), BF),
            jax.ShapeDtypeStruct((S, Dr), BF),
        ),
        scratch_shapes=[
            pltpu.VMEM((S, bn), F32),
            pltpu.VMEM((S, D), F32),
            pltpu.VMEM((S, D), F32),
            pltpu.VMEM((S, DCH), F32),
            pltpu.VMEM((S, DCH), F32),
            pltpu.VMEM((S, Dr), F32),
            pltpu.VMEM((S, DCH), BF),
            pltpu.VMEM((DCH, D), BF),
            pltpu.VMEM((DCH, D), BF),
            pltpu.SemaphoreType.DMA((2,)),
            pltpu.SemaphoreType.DMA((3,)),
            pltpu.SemaphoreType.DMA((3,)),
        ],
        compiler_params=pltpu.CompilerParams(collective_id=0),
    )(x, wdkv, wuk, wuv, wkr, wq)


def _attention(xm_bf, q, kr, cm, cp, wqr, wuk, wuv, wukp, wuvp):
    HG = 8
    CC = (((1,), (1,)), ((), ()))

    def body(x_ref, q_ref, kr_ref, cm_ref, cp_ref, wqr_ref,
             wuk_ref, wuv_ref, wukp_ref, wuvp_ref, o_ref,
             kg_ref, vg_ref):
        x_bf = x_ref[...]
        krm = kr_ref[...]
        qr_g = jnp.dot(x_bf, wqr_ref[...].astype(BF),
                       preferred_element_type=F32).astype(BF)
        cmv = cm_ref[...]
        cpv = cp_ref[...]
        kg_ref[...] = (
            jnp.dot(cmv, wuk_ref[...].astype(BF),
                    preferred_element_type=F32)
            + jnp.dot(cpv, wukp_ref[...], preferred_element_type=F32)
        ).astype(BF)
        vg_ref[...] = (
            jnp.dot(cmv, wuv_ref[...].astype(BF),
                    preferred_element_type=F32)
            + jnp.dot(cpv, wuvp_ref[...], preferred_element_type=F32)
        ).astype(BF)
        for i in range(HG):
            sl = slice(i * Dh, (i + 1) * Dh)
            s = lax.dot_general(q_ref[:, sl], kg_ref[:, sl], CC,
                                preferred_element_type=F32)
            s += lax.dot_general(qr_g[:, i * Dr:(i + 1) * Dr], krm, CC,
                                 preferred_element_type=F32)
            s *= SCALE
            mx = jnp.max(s, axis=-1, keepdims=True)
            p = jnp.exp(s - mx)
            den = jnp.sum(p, axis=-1, keepdims=True)
            o = jnp.dot(p.astype(BF), vg_ref[:, sl],
                        preferred_element_type=F32)
            o_ref[:, sl] = (o / den).astype(BF)

    return pl.pallas_call(
        body,
        grid=(H // HG,),
        in_specs=[
            pl.BlockSpec((S, D), lambda g: (0, 0)),
            pl.BlockSpec((S, HG * Dh), lambda g: (0, g)),
            pl.BlockSpec((S, Dr), lambda g: (0, 0)),
            pl.BlockSpec((S, DCH), lambda g: (0, 0)),
            pl.BlockSpec((S, DCH), lambda g: (0, 0)),
            pl.BlockSpec((D, HG * Dr), lambda g: (0, g)),
            pl.BlockSpec((DCH, HG * Dh), lambda g: (0, g)),
            pl.BlockSpec((DCH, HG * Dh), lambda g: (0, g)),
            pl.BlockSpec((DCH, HG * Dh), lambda g: (0, g)),
            pl.BlockSpec((DCH, HG * Dh), lambda g: (0, g)),
        ],
        out_specs=pl.BlockSpec((S, HG * Dh), lambda g: (0, g)),
        out_shape=jax.ShapeDtypeStruct((S, H * Dh), BF),
        scratch_shapes=[
            pltpu.VMEM((S, HG * Dh), BF),
            pltpu.VMEM((S, HG * Dh), BF),
        ],
    )(xm_bf, q, kr, cm, cp, wqr, wuk, wuv, wukp, wuvp)


def _wo_mm_ag(o_b, wo, bn=1024, bk=1024):
    NB = D // bn
    nk = D // bk

    def body(a_ref, w_ref, out_ref, acc_ref, sbuf, rby, rbx, rbd,
             sa, ra, sb, rb, sc, rc):
        ni = pl.program_id(0)
        ki = pl.program_id(1)
        my_x = lax.axis_index("x")
        my_y = lax.axis_index("y")
        ypeer = (my_x, 1 - my_y)
        xpeer = (1 - my_x, my_y)
        @pl.when(jnp.logical_and(ni == 0, ki == 0))
        def _():
            bar = pltpu.get_barrier_semaphore()
            for p in (ypeer, xpeer):
                pl.semaphore_signal(bar, inc=1, device_id=p,
                                    device_id_type=MESH)
            pl.semaphore_wait(bar, 2)

        @pl.when(ki == 0)
        def _():
            acc_ref[...] = jnp.zeros_like(acc_ref)

        acc_ref[...] += jnp.dot(a_ref[...], w_ref[...].astype(BF),
                                preferred_element_type=F32)

        def desc_a(j):
            jc = pl.ds(j * bn, bn)
            return pltpu.make_async_remote_copy(
                sbuf.at[:, jc], rby.at[:, jc], sa.at[j], ra.at[j],
                device_id=ypeer, device_id_type=MESH)

        def desc_b(j):
            jc = pl.ds(j * bn, bn)
            return pltpu.make_async_remote_copy(
                sbuf.at[:, jc], rbx.at[:, jc], sb.at[j], rb.at[j],
                device_id=xpeer, device_id_type=MESH)

        def desc_c(j):
            jc = pl.ds(j * bn, bn)
            if j % 2 == 0:
                return pltpu.make_async_remote_copy(
                    rby.at[:, jc], rbd.at[:, jc], sc.at[j], rc.at[j],
                    device_id=xpeer, device_id_type=MESH)
            return pltpu.make_async_remote_copy(
                rbx.at[:, jc], rbd.at[:, jc], sc.at[j], rc.at[j],
                device_id=ypeer, device_id_type=MESH)

        def fwd(j):
            (desc_a(j) if j % 2 == 0 else desc_b(j)).wait_recv()
            desc_c(j).start()

        @pl.when(ki == nk - 1)
        def _():
            cols = pl.ds(ni * bn, bn)
            blk = acc_ref[...]
            sbuf[:, cols] = blk.astype(BF)
            out_ref[pl.ds(2 * my_x + my_y, 1), :, cols] = blk[None]

        for jj in range(NB):
            @pl.when(jnp.logical_and(ni == jj, ki == nk - 1))
            def _(jj=jj):
                desc_a(jj).start()
                desc_b(jj).start()
                if jj > 0:
                    fwd(jj - 1)

        @pl.when(jnp.logical_and(ni == NB - 1, ki == nk - 1))
        def _():
            fwd(NB - 1)
            for j in range(NB):
                for d in (desc_a(j), desc_b(j), desc_c(j)):
                    d.wait_send()
                (desc_b(j) if j % 2 == 0 else desc_a(j)).wait_recv()
                desc_c(j).wait_recv()
            out_ref[pl.ds(2 * my_x + 1 - my_y, 1)] = (
                rby[...].astype(F32)[None])
            out_ref[pl.ds(2 * (1 - my_x) + my_y, 1)] = (
                rbx[...].astype(F32)[None])
            out_ref[pl.ds(2 * (1 - my_x) + 1 - my_y, 1)] = (
                rbd[...].astype(F32)[None])

    return pl.pallas_call(
        body,
        grid=(NB, nk),
        in_specs=[
            pl.BlockSpec((S, bk), lambda ni, ki: (0, ki)),
            pl.BlockSpec((bk, bn), lambda ni, ki: (ki, ni)),
        ],
        out_specs=pl.BlockSpec((B, S, D), lambda ni, ki: (0, 0, 0)),
        out_shape=jax.ShapeDtypeStruct((B, S, D), F32),
        scratch_shapes=[
            pltpu.VMEM((S, bn), F32),
            pltpu.VMEM((S, D), BF),
            pltpu.VMEM((S, D), BF),
            pltpu.VMEM((S, D), BF),
            pltpu.VMEM((S, D), BF),
            pltpu.SemaphoreType.DMA((NB,)),
            pltpu.SemaphoreType.DMA((NB,)),
            pltpu.SemaphoreType.DMA((NB,)),
            pltpu.SemaphoreType.DMA((NB,)),
            pltpu.SemaphoreType.DMA((NB,)),
            pltpu.SemaphoreType.DMA((NB,)),
        ],
        compiler_params=pltpu.CompilerParams(collective_id=1),
    )(o_b, wo)


def kernel(x, Wdkv, Wuk, Wuv, Wq, Wqr, Wkr, Wo):
    q, cm, cp, wukp, wuvp, xmbf, kr = _q_mm_exchange(
        x, Wdkv, Wuk, Wuv, Wkr, Wq)
    o = _attention(xmbf, q, kr, cm, cp, Wqr, Wuk, Wuv, wukp, wuvp)
    return _wo_mm_ag(o, Wo, bn=512, bk=2048)
